# baseline (speedup 1.0000x reference)
import functools

import jax
import jax.numpy as jnp
from jax import lax
from jax.experimental import pallas as pl
from jax.experimental.pallas import tpu as pltpu
from jax.experimental.pallas import tpu_sc as plsc

N = 10000
E = 320000
F = 128

NUM_CORES = 2
NUM_SUBCORES = 16
NUM_TILES = NUM_CORES * NUM_SUBCORES

EDGES_PER_TILE = E // NUM_TILES
CHUNK = 80
NCHUNKS = EDGES_PER_TILE // CHUNK
NPAD = 10240
ROWS_PER_TILE = NPAD // NUM_SUBCORES
ZROWS = 160

M_BLK = 1000


def _mm1_kernel(x_ref, w_ref, b_ref, o_ref):
    o_ref[...] = lax.dot_general(
        x_ref[...], w_ref[...], (((1,), (1,)), ((), ())),
        preferred_element_type=jnp.float32) + b_ref[...]


def _mm2_kernel(a_ref, w_ref, b_ref, o_ref):
    a = a_ref[0] + a_ref[1]
    o_ref[...] = lax.dot_general(
        a, w_ref[...], (((1,), (1,)), ((), ())),
        preferred_element_type=jnp.float32) + b_ref[...]


NBUF = 3
IDX_BITS = 14
IDX_MASK = (1 << IDX_BITS) - 1


def _widen(v32):
    lo = lax.bitcast_convert_type(lax.shift_left(v32, 16), jnp.float32)
    hi = lax.bitcast_convert_type(v32 & jnp.int32(-65536), jnp.float32)
    return lo, hi


def _scale_rows(rows_b, w_all, ck):
    def _one(k, wk):
        for j in range(F // 16):
            rows_b[k, pl.ds(j * 16, 16)] = rows_b[k, pl.ds(j * 16, 16)] * wk

    wbase = ck * (CHUNK // 2)
    for t in range(3):
        wv32 = w_all[pl.ds(wbase + t * 16, 16)]
        wlo, whi = _widen(wv32)
        for m in range(16 if t < 2 else 8):
            _one(32 * t + 2 * m, wlo[m])
            _one(32 * t + 2 * m + 1, whi[m])


def _sc_body(h_hbm, packed_hbm, ew_hbm, out_hbm,
             packed_all, w_all, sidx_b, didx_b, rows, acc, gsems, ssems):
    c = lax.axis_index("c")
    s = lax.axis_index("s")
    wid = c * NUM_SUBCORES + s

    def _unpack(ck, b):
        for g in range(CHUNK // 16):
            pv = packed_all[pl.ds(ck * CHUNK + g * 16, 16)]
            sidx_b[b, pl.ds(g * 16, 16)] = pv & IDX_MASK
            didx_b[b, pl.ds(g * 16, 16)] = lax.shift_right_logical(pv, IDX_BITS)

    def _fire_gather(ck, b):
        _unpack(ck, b)
        pltpu.async_copy(h_hbm.at[sidx_b.at[b]], rows.at[b], gsems.at[b])

    def _wait_gather(b):
        pltpu.make_async_copy(h_hbm.at[sidx_b.at[b]], rows.at[b],
                              gsems.at[b]).wait()

    pltpu.sync_copy(packed_hbm.at[pl.ds(wid * EDGES_PER_TILE, EDGES_PER_TILE)],
                    packed_all)
    pltpu.sync_copy(ew_hbm.at[pl.ds(wid * (EDGES_PER_TILE // 2),
                                    EDGES_PER_TILE // 2)],
                    w_all.at[pl.ds(0, EDGES_PER_TILE // 2)])

    def _zero_body(i, _):
        z = jnp.zeros((16,), jnp.float32)
        for j in range(F // 16):
            rows[0, i, pl.ds(j * 16, 16)] = z
        return 0
    lax.fori_loop(0, CHUNK, _zero_body, 0)
    for t in range(ROWS_PER_TILE // CHUNK):
        pltpu.sync_copy(rows.at[0],
                        acc.at[pl.ds(s * ROWS_PER_TILE + t * CHUNK, CHUNK)])
    plsc.subcore_barrier()

    for b in range(NBUF):
        _fire_gather(b, b)

    def _iter(i, _):
        c0 = i * NBUF
        for b in range(NBUF):
            _wait_gather(b)
            _scale_rows(rows.at[b], w_all, c0 + b)
            pltpu.async_copy(rows.at[b], acc.at[didx_b.at[b]], ssems.at[b],
                             add=True)
        for b in range(NBUF):
            ck = c0 + b
            pltpu.make_async_copy(rows.at[b], acc.at[didx_b.at[b]],
                                  ssems.at[b]).wait()

            @pl.when(ck + NBUF < NCHUNKS)
            def _():
                _fire_gather(ck + NBUF, b)
        return 0
    lax.fori_loop(0, NCHUNKS // NBUF, _iter, 0)

    for ck in range(NBUF * (NCHUNKS // NBUF), NCHUNKS):
        b = ck % NBUF
        _wait_gather(b)
        _scale_rows(rows.at[b], w_all, ck)
        pltpu.sync_copy(rows.at[b], acc.at[didx_b.at[b]], add=True)
    plsc.subcore_barrier()

    for t in range(ROWS_PER_TILE // CHUNK):
        r0 = s * ROWS_PER_TILE + t * CHUNK
        pltpu.sync_copy(acc.at[pl.ds(r0, CHUNK)], rows.at[0])
        pltpu.sync_copy(rows.at[0], out_hbm.at[c, pl.ds(r0, CHUNK)])


_sc_scatter = functools.partial(
    pl.kernel,
    mesh=plsc.VectorSubcoreMesh(core_axis_name="c", subcore_axis_name="s"),
    out_type=jax.ShapeDtypeStruct((NUM_CORES, NPAD, F), jnp.float32),
    scratch_types=[
        pltpu.VMEM((EDGES_PER_TILE,), jnp.int32),
        pltpu.VMEM((EDGES_PER_TILE // 2 + 16,), jnp.int32),
        pltpu.VMEM((NBUF, CHUNK), jnp.int32),
        pltpu.VMEM((NBUF, CHUNK), jnp.int32),
        pltpu.VMEM((NBUF, CHUNK, F), jnp.float32),
        pltpu.VMEM_SHARED((NPAD, F), jnp.float32),
        pltpu.SemaphoreType.DMA((NBUF,)),
        pltpu.SemaphoreType.DMA((NBUF,)),
    ],
)(_sc_body)


def kernel(x, edge_index, edge_weight, W_w, b_w, W_lin, b_lin):
    src = edge_index[0].astype(jnp.int32)
    dst = edge_index[1].astype(jnp.int32)
    packed = src | (dst << IDX_BITS)
    wpk = lax.bitcast_convert_type(
        edge_weight.astype(jnp.bfloat16).reshape(E // 2, 2), jnp.int32)

    h = pl.pallas_call(
        _mm1_kernel,
        grid=(N // M_BLK,),
        in_specs=[
            pl.BlockSpec((M_BLK, F), lambda i: (i, 0)),
            pl.BlockSpec((F, F), lambda i: (0, 0)),
            pl.BlockSpec((1, F), lambda i: (0, 0)),
        ],
        out_specs=pl.BlockSpec((M_BLK, F), lambda i: (i, 0)),
        out_shape=jax.ShapeDtypeStruct((N, F), jnp.float32),
    )(x, W_w, b_w.reshape(1, F))

    partials = _sc_scatter(h, packed, wpk)

    out = pl.pallas_call(
        _mm2_kernel,
        grid=(N // M_BLK,),
        in_specs=[
            pl.BlockSpec((NUM_CORES, M_BLK, F), lambda i: (0, i, 0)),
            pl.BlockSpec((F, F), lambda i: (0, 0)),
            pl.BlockSpec((1, F), lambda i: (0, 0)),
        ],
        out_specs=pl.BlockSpec((M_BLK, F), lambda i: (i, 0)),
        out_shape=jax.ShapeDtypeStruct((N, F), jnp.float32),
    )(partials, W_lin, b_lin.reshape(1, F))
    return out


# speedup vs baseline: 1.1926x; 1.1926x over previous
import functools

import jax
import jax.numpy as jnp
from jax import lax
from jax.experimental import pallas as pl
from jax.experimental.pallas import tpu as pltpu
from jax.experimental.pallas import tpu_sc as plsc

N = 10000
E = 320000
F = 128

NUM_CORES = 2
NUM_SUBCORES = 16
NUM_TILES = NUM_CORES * NUM_SUBCORES

EDGES_PER_TILE = E // NUM_TILES
CHUNK = 80
NCHUNKS = EDGES_PER_TILE // CHUNK
NPAD = 10240
ROWS_PER_TILE = NPAD // NUM_SUBCORES
ZROWS = 160

M_BLK = 1000


def _mm1_kernel(x_ref, w_ref, b_ref, o_ref):
    o_ref[...] = lax.dot_general(
        x_ref[...], w_ref[...], (((1,), (1,)), ((), ())),
        preferred_element_type=jnp.float32) + b_ref[...]


def _mm2_kernel(a_ref, w_ref, b_ref, o_ref):
    a = a_ref[0] + a_ref[1]
    o_ref[...] = lax.dot_general(
        a, w_ref[...], (((1,), (1,)), ((), ())),
        preferred_element_type=jnp.float32) + b_ref[...]


NBUF = 2
IDX_BITS = 14
IDX_MASK = (1 << IDX_BITS) - 1


def _widen(v32):
    lo = lax.bitcast_convert_type(lax.shift_left(v32, 16), jnp.float32)
    hi = lax.bitcast_convert_type(v32 & jnp.int32(-65536), jnp.float32)
    return lo, hi


def _scale_rows(rows_b, w_all, ck):
    def _one(k, wk):
        for j in range(F // 16):
            rows_b[k, pl.ds(j * 16, 16)] = rows_b[k, pl.ds(j * 16, 16)] * wk

    wbase = ck * (CHUNK // 2)
    for t in range(3):
        wv32 = w_all[pl.ds(wbase + t * 16, 16)]
        wlo, whi = _widen(wv32)
        for m in range(16 if t < 2 else 8):
            _one(32 * t + 2 * m, wlo[m])
            _one(32 * t + 2 * m + 1, whi[m])


def _sc_body(h_hbm, packed_hbm, ew_hbm, out_hbm,
             packed_all, w_all, sidx_b, didx_b, rows, acc, gsems, ssems):
    c = lax.axis_index("c")
    s = lax.axis_index("s")
    wid = c * NUM_SUBCORES + s

    def _unpack(ck, b):
        for g in range(CHUNK // 16):
            pv = packed_all[pl.ds(ck * CHUNK + g * 16, 16)]
            sidx_b[b, pl.ds(g * 16, 16)] = pv & IDX_MASK
            didx_b[b, pl.ds(g * 16, 16)] = lax.shift_right_logical(pv, IDX_BITS)

    def _fire_gather(ck, b):
        _unpack(ck, b)
        pltpu.async_copy(h_hbm.at[sidx_b.at[b]], rows.at[b], gsems.at[b])

    def _wait_gather(b):
        pltpu.make_async_copy(h_hbm.at[sidx_b.at[b]], rows.at[b],
                              gsems.at[b]).wait()

    pltpu.sync_copy(packed_hbm.at[pl.ds(wid * EDGES_PER_TILE, EDGES_PER_TILE)],
                    packed_all)
    pltpu.sync_copy(ew_hbm.at[pl.ds(wid * (EDGES_PER_TILE // 2),
                                    EDGES_PER_TILE // 2)],
                    w_all.at[pl.ds(0, EDGES_PER_TILE // 2)])

    def _zero_body(i, _):
        z = jnp.zeros((16,), jnp.float32)
        for j in range(F // 16):
            rows[0, i, pl.ds(j * 16, 16)] = z
        return 0
    lax.fori_loop(0, CHUNK, _zero_body, 0)
    for t in range(ROWS_PER_TILE // CHUNK):
        pltpu.sync_copy(rows.at[0],
                        acc.at[pl.ds(s * ROWS_PER_TILE + t * CHUNK, CHUNK)])
    plsc.subcore_barrier()

    for b in range(NBUF):
        _fire_gather(b, b)

    def _iter(i, _):
        c0 = i * NBUF
        for b in range(NBUF):
            _wait_gather(b)
            _scale_rows(rows.at[b], w_all, c0 + b)
            pltpu.async_copy(rows.at[b], acc.at[didx_b.at[b]], ssems.at[b],
                             add=True)
        for b in range(NBUF):
            ck = c0 + b
            pltpu.make_async_copy(rows.at[b], acc.at[didx_b.at[b]],
                                  ssems.at[b]).wait()

            @pl.when(ck + NBUF < NCHUNKS)
            def _():
                _fire_gather(ck + NBUF, b)
        return 0
    lax.fori_loop(0, NCHUNKS // NBUF, _iter, 0)

    for ck in range(NBUF * (NCHUNKS // NBUF), NCHUNKS):
        b = ck % NBUF
        _wait_gather(b)
        _scale_rows(rows.at[b], w_all, ck)
        pltpu.sync_copy(rows.at[b], acc.at[didx_b.at[b]], add=True)
    plsc.subcore_barrier()

    for t in range(ROWS_PER_TILE // CHUNK):
        r0 = s * ROWS_PER_TILE + t * CHUNK
        pltpu.sync_copy(acc.at[pl.ds(r0, CHUNK)], rows.at[0])
        pltpu.sync_copy(rows.at[0], out_hbm.at[c, pl.ds(r0, CHUNK)])


_sc_scatter = functools.partial(
    pl.kernel,
    mesh=plsc.VectorSubcoreMesh(core_axis_name="c", subcore_axis_name="s"),
    out_type=jax.ShapeDtypeStruct((NUM_CORES, NPAD, F), jnp.float32),
    scratch_types=[
        pltpu.VMEM((EDGES_PER_TILE,), jnp.int32),
        pltpu.VMEM((EDGES_PER_TILE // 2 + 16,), jnp.int32),
        pltpu.VMEM((NBUF, CHUNK), jnp.int32),
        pltpu.VMEM((NBUF, CHUNK), jnp.int32),
        pltpu.VMEM((NBUF, CHUNK, F), jnp.float32),
        pltpu.VMEM_SHARED((NPAD, F), jnp.float32),
        pltpu.SemaphoreType.DMA((NBUF,)),
        pltpu.SemaphoreType.DMA((NBUF,)),
    ],
)(_sc_body)


def kernel(x, edge_index, edge_weight, W_w, b_w, W_lin, b_lin):
    src = edge_index[0].astype(jnp.int32)
    dst = edge_index[1].astype(jnp.int32)
    packed = src | (dst << IDX_BITS)
    wpk = lax.bitcast_convert_type(
        edge_weight.astype(jnp.bfloat16).reshape(E // 2, 2), jnp.int32)

    h = pl.pallas_call(
        _mm1_kernel,
        grid=(N // M_BLK,),
        in_specs=[
            pl.BlockSpec((M_BLK, F), lambda i: (i, 0)),
            pl.BlockSpec((F, F), lambda i: (0, 0)),
            pl.BlockSpec((1, F), lambda i: (0, 0)),
        ],
        out_specs=pl.BlockSpec((M_BLK, F), lambda i: (i, 0)),
        out_shape=jax.ShapeDtypeStruct((N, F), jnp.float32),
    )(x, W_w, b_w.reshape(1, F))

    partials = _sc_scatter(h, packed, wpk)

    out = pl.pallas_call(
        _mm2_kernel,
        grid=(N // M_BLK,),
        in_specs=[
            pl.BlockSpec((NUM_CORES, M_BLK, F), lambda i: (0, i, 0)),
            pl.BlockSpec((F, F), lambda i: (0, 0)),
            pl.BlockSpec((1, F), lambda i: (0, 0)),
        ],
        out_specs=pl.BlockSpec((M_BLK, F), lambda i: (i, 0)),
        out_shape=jax.ShapeDtypeStruct((N, F), jnp.float32),
    )(partials, W_lin, b_lin.reshape(1, F))
    return out


# speedup vs baseline: 1.5141x; 1.2696x over previous
import functools

import jax
import jax.numpy as jnp
from jax import lax
from jax.experimental import pallas as pl
from jax.experimental.pallas import tpu as pltpu
from jax.experimental.pallas import tpu_sc as plsc

N = 10000
E = 320000
F = 128

NUM_CORES = 2
NUM_SUBCORES = 16
NUM_TILES = NUM_CORES * NUM_SUBCORES

EDGES_PER_TILE = E // NUM_TILES
CHUNK = 80
NCHUNKS = EDGES_PER_TILE // CHUNK
NPAD = 10240
ROWS_PER_TILE = NPAD // NUM_SUBCORES
ZROWS = 160

M_BLK = 1000


def _mm1_kernel(x_ref, w_ref, b_ref, o_ref):
    o_ref[...] = lax.dot_general(
        x_ref[...], w_ref[...], (((1,), (1,)), ((), ())),
        preferred_element_type=jnp.float32) + b_ref[...]


def _mm2_kernel(a_ref, w_ref, b_ref, o_ref):
    a = a_ref[0] + a_ref[1]
    o_ref[...] = lax.dot_general(
        a, w_ref[...], (((1,), (1,)), ((), ())),
        preferred_element_type=jnp.float32) + b_ref[...]


NBUF = 3
IDX_BITS = 14
IDX_MASK = (1 << IDX_BITS) - 1


def _scale_rows(rows_b, wb, ck):
    for g in range(CHUNK // 16):
        wv = wb[pl.ds(g * 16, 16)]
        for l in range(16):
            k = g * 16 + l
            wk = wv[l]
            for j in range(F // 16):
                rows_b[k, pl.ds(j * 16, 16)] = rows_b[k, pl.ds(j * 16, 16)] * wk


def _sc_body(h_hbm, packed_hbm, ew_hbm, out_hbm,
             packed_all, wbuf, sidx_b, didx_b, rows, acc, gsems, ssems):
    c = lax.axis_index("c")
    s = lax.axis_index("s")
    wid = c * NUM_SUBCORES + s

    def _unpack(ck, b):
        for g in range(CHUNK // 16):
            pv = packed_all[pl.ds(ck * CHUNK + g * 16, 16)]
            sidx_b[b, pl.ds(g * 16, 16)] = pv & IDX_MASK
            didx_b[b, pl.ds(g * 16, 16)] = lax.shift_right_logical(pv, IDX_BITS)

    def _fire_gather(ck, b):
        _unpack(ck, b)
        pltpu.async_copy(ew_hbm.at[pl.ds(wid * EDGES_PER_TILE + ck * CHUNK,
                                         CHUNK)],
                         wbuf.at[b], gsems.at[b])
        pltpu.async_copy(h_hbm.at[sidx_b.at[b]], rows.at[b], gsems.at[b])

    def _wait_gather(b):
        pltpu.make_async_copy(ew_hbm.at[pl.ds(0, CHUNK)], wbuf.at[b],
                              gsems.at[b]).wait()
        pltpu.make_async_copy(h_hbm.at[sidx_b.at[b]], rows.at[b],
                              gsems.at[b]).wait()

    pltpu.sync_copy(packed_hbm.at[pl.ds(wid * EDGES_PER_TILE, EDGES_PER_TILE)],
                    packed_all)

    def _zero_body(i, _):
        z = jnp.zeros((16,), jnp.float32)
        for j in range(F // 16):
            rows[0, i, pl.ds(j * 16, 16)] = z
        return 0
    lax.fori_loop(0, CHUNK, _zero_body, 0)
    for t in range(ROWS_PER_TILE // CHUNK):
        pltpu.sync_copy(rows.at[0],
                        acc.at[pl.ds(s * ROWS_PER_TILE + t * CHUNK, CHUNK)])
    plsc.subcore_barrier()

    for b in range(NBUF):
        _fire_gather(b, b)

    def _iter(i, _):
        c0 = i * NBUF
        for b in range(NBUF):
            _wait_gather(b)
            _scale_rows(rows.at[b], wbuf.at[b], c0 + b)
            pltpu.async_copy(rows.at[b], acc.at[didx_b.at[b]], ssems.at[b],
                             add=True)
        for b in range(NBUF):
            ck = c0 + b
            pltpu.make_async_copy(rows.at[b], acc.at[didx_b.at[b]],
                                  ssems.at[b]).wait()

            @pl.when(ck + NBUF < NCHUNKS)
            def _():
                _fire_gather(ck + NBUF, b)
        return 0
    lax.fori_loop(0, NCHUNKS // NBUF, _iter, 0)

    for ck in range(NBUF * (NCHUNKS // NBUF), NCHUNKS):
        b = ck % NBUF
        _wait_gather(b)
        _scale_rows(rows.at[b], wbuf.at[b], ck)
        pltpu.sync_copy(rows.at[b], acc.at[didx_b.at[b]], add=True)
    plsc.subcore_barrier()

    for t in range(ROWS_PER_TILE // CHUNK):
        r0 = s * ROWS_PER_TILE + t * CHUNK
        pltpu.sync_copy(acc.at[pl.ds(r0, CHUNK)], rows.at[0])
        pltpu.sync_copy(rows.at[0], out_hbm.at[c, pl.ds(r0, CHUNK)])


_sc_scatter = functools.partial(
    pl.kernel,
    mesh=plsc.VectorSubcoreMesh(core_axis_name="c", subcore_axis_name="s"),
    out_type=jax.ShapeDtypeStruct((NUM_CORES, NPAD, F), jnp.float32),
    scratch_types=[
        pltpu.VMEM((EDGES_PER_TILE,), jnp.int32),
        pltpu.VMEM((NBUF, CHUNK), jnp.float32),
        pltpu.VMEM((NBUF, CHUNK), jnp.int32),
        pltpu.VMEM((NBUF, CHUNK), jnp.int32),
        pltpu.VMEM((NBUF, CHUNK, F), jnp.float32),
        pltpu.VMEM_SHARED((NPAD, F), jnp.float32),
        pltpu.SemaphoreType.DMA((NBUF,)),
        pltpu.SemaphoreType.DMA((NBUF,)),
    ],
)(_sc_body)


def kernel(x, edge_index, edge_weight, W_w, b_w, W_lin, b_lin):
    src = edge_index[0].astype(jnp.int32)
    dst = edge_index[1].astype(jnp.int32)
    packed = src | (dst << IDX_BITS)
    ew = edge_weight.astype(jnp.float32)

    h = pl.pallas_call(
        _mm1_kernel,
        grid=(N // M_BLK,),
        in_specs=[
            pl.BlockSpec((M_BLK, F), lambda i: (i, 0)),
            pl.BlockSpec((F, F), lambda i: (0, 0)),
            pl.BlockSpec((1, F), lambda i: (0, 0)),
        ],
        out_specs=pl.BlockSpec((M_BLK, F), lambda i: (i, 0)),
        out_shape=jax.ShapeDtypeStruct((N, F), jnp.float32),
    )(x, W_w, b_w.reshape(1, F))

    partials = _sc_scatter(h, packed, ew)

    out = pl.pallas_call(
        _mm2_kernel,
        grid=(N // M_BLK,),
        in_specs=[
            pl.BlockSpec((NUM_CORES, M_BLK, F), lambda i: (0, i, 0)),
            pl.BlockSpec((F, F), lambda i: (0, 0)),
            pl.BlockSpec((1, F), lambda i: (0, 0)),
        ],
        out_specs=pl.BlockSpec((M_BLK, F), lambda i: (i, 0)),
        out_shape=jax.ShapeDtypeStruct((N, F), jnp.float32),
    )(partials, W_lin, b_lin.reshape(1, F))
    return out


# speedup vs baseline: 1.9721x; 1.3025x over previous
import functools

import jax
import jax.numpy as jnp
from jax import lax
from jax.experimental import pallas as pl
from jax.experimental.pallas import tpu as pltpu
from jax.experimental.pallas import tpu_sc as plsc

N = 10000
E = 320000
F = 128

NUM_CORES = 2
NUM_SUBCORES = 16
NUM_TILES = NUM_CORES * NUM_SUBCORES

EDGES_PER_TILE = E // NUM_TILES
CHUNK = 80
NCHUNKS = EDGES_PER_TILE // CHUNK
NPAD = 10240
ROWS_PER_TILE = NPAD // NUM_SUBCORES
ZROWS = 160

M_BLK = 1000


def _mm1_kernel(x_ref, w_ref, b_ref, o_ref):
    o_ref[...] = lax.dot_general(
        x_ref[...], w_ref[...], (((1,), (1,)), ((), ())),
        preferred_element_type=jnp.float32) + b_ref[...]


def _mm2_kernel(a_ref, w_ref, b_ref, o_ref):
    a = a_ref[0] + a_ref[1]
    o_ref[...] = lax.dot_general(
        a, w_ref[...], (((1,), (1,)), ((), ())),
        preferred_element_type=jnp.float32) + b_ref[...]


NBUF = 2
IDX_BITS = 14
IDX_MASK = (1 << IDX_BITS) - 1


def _scale_rows(rows_b, w_all, ck):
    for g in range(CHUNK // 16):
        wv = w_all[pl.ds(ck * CHUNK + g * 16, 16)]
        for l in range(16):
            k = g * 16 + l
            wk = wv[l]
            for j in range(F // 16):
                rows_b[k, pl.ds(j * 16, 16)] = rows_b[k, pl.ds(j * 16, 16)] * wk


def _sc_body(h_hbm, packed_hbm, ew_hbm, out_hbm,
             packed_all, w_all, sidx_b, didx_b, rows, acc, gsems, ssems):
    c = lax.axis_index("c")
    s = lax.axis_index("s")
    wid = c * NUM_SUBCORES + s

    def _unpack(ck, b):
        for g in range(CHUNK // 16):
            pv = packed_all[pl.ds(ck * CHUNK + g * 16, 16)]
            sidx_b[b, pl.ds(g * 16, 16)] = pv & IDX_MASK
            didx_b[b, pl.ds(g * 16, 16)] = lax.shift_right_logical(pv, IDX_BITS)

    def _fire_gather(ck, b):
        _unpack(ck, b)
        pltpu.async_copy(h_hbm.at[sidx_b.at[b]], rows.at[b], gsems.at[b])

    def _wait_gather(b):
        pltpu.make_async_copy(h_hbm.at[sidx_b.at[b]], rows.at[b],
                              gsems.at[b]).wait()

    pltpu.sync_copy(packed_hbm.at[pl.ds(wid * EDGES_PER_TILE, EDGES_PER_TILE)],
                    packed_all)
    pltpu.sync_copy(ew_hbm.at[pl.ds(wid * EDGES_PER_TILE, EDGES_PER_TILE)],
                    w_all)

    def _zero_body(i, _):
        z = jnp.zeros((16,), jnp.float32)
        for j in range(F // 16):
            rows[0, i, pl.ds(j * 16, 16)] = z
        return 0
    lax.fori_loop(0, CHUNK, _zero_body, 0)
    for t in range(ROWS_PER_TILE // CHUNK):
        pltpu.sync_copy(rows.at[0],
                        acc.at[pl.ds(s * ROWS_PER_TILE + t * CHUNK, CHUNK)])
    plsc.subcore_barrier()

    for b in range(NBUF):
        _fire_gather(b, b)

    def _iter(i, _):
        c0 = i * NBUF
        for b in range(NBUF):
            _wait_gather(b)
            _scale_rows(rows.at[b], w_all, c0 + b)
            pltpu.async_copy(rows.at[b], acc.at[didx_b.at[b]], ssems.at[b],
                             add=True)
        for b in range(NBUF):
            ck = c0 + b
            pltpu.make_async_copy(rows.at[b], acc.at[didx_b.at[b]],
                                  ssems.at[b]).wait()

            @pl.when(ck + NBUF < NCHUNKS)
            def _():
                _fire_gather(ck + NBUF, b)
        return 0
    lax.fori_loop(0, NCHUNKS // NBUF, _iter, 0)

    for ck in range(NBUF * (NCHUNKS // NBUF), NCHUNKS):
        b = ck % NBUF
        _wait_gather(b)
        _scale_rows(rows.at[b], w_all, ck)
        pltpu.sync_copy(rows.at[b], acc.at[didx_b.at[b]], add=True)
    plsc.subcore_barrier()

    for t in range(ROWS_PER_TILE // CHUNK):
        r0 = s * ROWS_PER_TILE + t * CHUNK
        pltpu.sync_copy(acc.at[pl.ds(r0, CHUNK)], rows.at[0])
        pltpu.sync_copy(rows.at[0], out_hbm.at[c, pl.ds(r0, CHUNK)])


_sc_scatter = functools.partial(
    pl.kernel,
    mesh=plsc.VectorSubcoreMesh(core_axis_name="c", subcore_axis_name="s"),
    out_type=jax.ShapeDtypeStruct((NUM_CORES, NPAD, F), jnp.float32),
    scratch_types=[
        pltpu.VMEM((EDGES_PER_TILE,), jnp.int32),
        pltpu.VMEM((EDGES_PER_TILE,), jnp.float32),
        pltpu.VMEM((NBUF, CHUNK), jnp.int32),
        pltpu.VMEM((NBUF, CHUNK), jnp.int32),
        pltpu.VMEM((NBUF, CHUNK, F), jnp.float32),
        pltpu.VMEM_SHARED((NPAD, F), jnp.float32),
        pltpu.SemaphoreType.DMA((NBUF,)),
        pltpu.SemaphoreType.DMA((NBUF,)),
    ],
)(_sc_body)


def kernel(x, edge_index, edge_weight, W_w, b_w, W_lin, b_lin):
    src = edge_index[0].astype(jnp.int32)
    dst = edge_index[1].astype(jnp.int32)
    packed = src | (dst << IDX_BITS)
    ew = edge_weight.astype(jnp.float32)

    h = pl.pallas_call(
        _mm1_kernel,
        grid=(N // M_BLK,),
        in_specs=[
            pl.BlockSpec((M_BLK, F), lambda i: (i, 0)),
            pl.BlockSpec((F, F), lambda i: (0, 0)),
            pl.BlockSpec((1, F), lambda i: (0, 0)),
        ],
        out_specs=pl.BlockSpec((M_BLK, F), lambda i: (i, 0)),
        out_shape=jax.ShapeDtypeStruct((N, F), jnp.float32),
    )(x, W_w, b_w.reshape(1, F))

    partials = _sc_scatter(h, packed, ew)

    out = pl.pallas_call(
        _mm2_kernel,
        grid=(N // M_BLK,),
        in_specs=[
            pl.BlockSpec((NUM_CORES, M_BLK, F), lambda i: (0, i, 0)),
            pl.BlockSpec((F, F), lambda i: (0, 0)),
            pl.BlockSpec((1, F), lambda i: (0, 0)),
        ],
        out_specs=pl.BlockSpec((M_BLK, F), lambda i: (i, 0)),
        out_shape=jax.ShapeDtypeStruct((N, F), jnp.float32),
    )(partials, W_lin, b_lin.reshape(1, F))
    return out
